# baseline (speedup 1.0000x reference)
import functools

import jax
import jax.numpy as jnp
from jax import lax
from jax.experimental import pallas as pl
from jax.experimental.pallas import tpu as pltpu
from jax.experimental.pallas import tpu_sc as plsc

GLOBAL_MEAN = 3.5
LANES = 16


def _make_sc_kernel(batch, embed_dim, num_workers, interpret=False):
    n = batch // num_workers
    ngroups = n // LANES

    mesh = plsc.VectorSubcoreMesh(
        core_axis_name="c", subcore_axis_name="s",
        num_cores=2, num_subcores=16)
    num_cores = mesh.num_cores

    @functools.partial(
        pl.kernel,
        out_type=jax.ShapeDtypeStruct((batch,), jnp.float32),
        mesh=mesh,
        scratch_types=[
            pltpu.VMEM((n,), jnp.int32),
            pltpu.VMEM((n,), jnp.int32),
            pltpu.VMEM((n, embed_dim), jnp.float32),
            pltpu.VMEM((n, embed_dim), jnp.float32),
            pltpu.VMEM((n,), jnp.float32),
            pltpu.VMEM((n,), jnp.float32),
            pltpu.VMEM((n,), jnp.float32),
            pltpu.SemaphoreType.DMA,
            pltpu.SemaphoreType.DMA,
            pltpu.SemaphoreType.DMA,
            pltpu.SemaphoreType.DMA,
        ],
        compiler_params=pltpu.CompilerParams(
            needs_layout_passes=False, use_tc_tiling_on_sc=False),
        interpret=interpret,
    )
    def sc_kernel(u_hbm, i_hbm, p_hbm, q_hbm, bu_hbm, bi_hbm, out_hbm,
                  uidx, iidx, prow, qrow, bu_v, bi_v, out_v,
                  sem0, sem1, sem2, sem3):
        wid = lax.axis_index("s") * num_cores + lax.axis_index("c")
        base = wid * n

        pltpu.sync_copy(u_hbm.at[pl.ds(base, n)], uidx)
        pltpu.sync_copy(i_hbm.at[pl.ds(base, n)], iidx)

        for k in range(ngroups):
            off = k * LANES
            uidx[pl.ds(off, LANES)] = jnp.maximum(
                uidx[pl.ds(off, LANES)] - 1, 0)
            iidx[pl.ds(off, LANES)] = jnp.maximum(
                iidx[pl.ds(off, LANES)] - 1, 0)

        cp0 = pltpu.async_copy(p_hbm.at[uidx], prow, sem0)
        cp1 = pltpu.async_copy(q_hbm.at[iidx], qrow, sem1)
        cp2 = pltpu.async_copy(bu_hbm.at[uidx], bu_v, sem2)
        cp3 = pltpu.async_copy(bi_hbm.at[iidx], bi_v, sem3)
        cp2.wait()
        cp3.wait()
        cp0.wait()
        cp1.wait()

        nchunks = embed_dim // LANES
        lane = lax.iota(jnp.int32, LANES)

        def group_body(g, carry):
            goff = pl.multiple_of(g * LANES, LANES)
            acc = jnp.zeros((LANES,), jnp.float32)
            for r in range(LANES):
                b = goff + r
                prod = prow[b, pl.ds(0, LANES)] * qrow[b, pl.ds(0, LANES)]
                for c in range(1, nchunks):
                    off = c * LANES
                    prod = prod + (prow[b, pl.ds(off, LANES)] *
                                   qrow[b, pl.ds(off, LANES)])
                s = jnp.sum(prod)
                acc = jnp.where(lane == r, s, acc)
            acc = acc + bu_v[pl.ds(goff, LANES)] + bi_v[pl.ds(goff, LANES)]
            out_v[pl.ds(goff, LANES)] = acc + GLOBAL_MEAN
            return carry

        lax.fori_loop(0, ngroups, group_body, 0)

        pltpu.sync_copy(out_v, out_hbm.at[pl.ds(base, n)])

    return sc_kernel


@jax.jit
def kernel(x, P, Q, B_U, B_I):
    batch = x.shape[1]
    embed_dim = P.shape[1]
    sc = _make_sc_kernel(batch, embed_dim, num_workers=32)
    return sc(x[0], x[1], P, Q, B_U.reshape(-1), B_I.reshape(-1))
